# Initial kernel scaffold; baseline (speedup 1.0000x reference)
import jax
import jax.numpy as jnp
from jax.experimental import pallas as pl


def kernel(ent_embedding, rel_embedding, current_triples, corrupted_triples):
    raise NotImplementedError("write your pallas kernel here")


# speedup vs baseline: 2.8314x; 2.8314x over previous
import functools

import jax
import jax.numpy as jnp
from jax import lax
from jax.experimental import pallas as pl
from jax.experimental.pallas import tpu as pltpu
from jax.experimental.pallas import tpu_sc as plsc

_BATCH = 4096
_DIM = 128
_L = 16

_info = plsc.get_sparse_core_info()
_NC = _info.num_cores
_NS = _info.num_subcores
_NW = _NC * _NS
_NB = _BATCH // _NW
_CH = _DIM // _L


def _sqrt_v(x):
    i = plsc.bitcast(x, jnp.int32)
    i = jnp.int32(0x5F3759DF) - lax.shift_right_logical(i, 1)
    z = plsc.bitcast(i, jnp.float32)
    for _ in range(3):
        z = z * (1.5 - 0.5 * x * z * z)
    return x * z


def _relu_bcast(s):
    v = jnp.broadcast_to(s, (_L,))
    return jnp.maximum(v, 0.0)


def _tec_body(ent_hbm, rel_hbm, h_hbm, r_hbm, t_hbm, hc_hbm, rc_hbm, tc_hbm,
              out_hbm,
              ih, ir, it, ihc, irc, itc,
              gh, gr, gt, ghc, grc, gtc,
              part, sem):
    wid = lax.axis_index("s") * _NC + lax.axis_index("c")
    base = wid * _NB

    pltpu.sync_copy(h_hbm.at[pl.ds(base, _NB)], ih)
    pltpu.sync_copy(r_hbm.at[pl.ds(base, _NB)], ir)
    pltpu.sync_copy(t_hbm.at[pl.ds(base, _NB)], it)
    pltpu.sync_copy(hc_hbm.at[pl.ds(base, _NB)], ihc)
    pltpu.sync_copy(rc_hbm.at[pl.ds(base, _NB)], irc)
    pltpu.sync_copy(tc_hbm.at[pl.ds(base, _NB)], itc)

    c0 = pltpu.async_copy(ent_hbm.at[ih], gh, sem)
    c1 = pltpu.async_copy(rel_hbm.at[ir], gr, sem)
    c2 = pltpu.async_copy(ent_hbm.at[it], gt, sem)
    c3 = pltpu.async_copy(ent_hbm.at[ihc], ghc, sem)
    c4 = pltpu.async_copy(rel_hbm.at[irc], grc, sem)
    c5 = pltpu.async_copy(ent_hbm.at[itc], gtc, sem)
    c0.wait(); c1.wait(); c2.wait(); c3.wait(); c4.wait(); c5.wait()

    zero = jnp.zeros((_L,), jnp.float32)

    def body(b, carry):
        loss_a, ent_a, rel_a = carry
        ad0 = zero; ad1 = zero
        ah = zero; at = zero; ahc = zero; atc = zero
        ar = zero; arc = zero
        for c in range(_CH):
            sl = pl.ds(c * _L, _L)
            hv = gh[b, sl]; rv = gr[b, sl]; tv = gt[b, sl]
            hv2 = ghc[b, sl]; rv2 = grc[b, sl]; tv2 = gtc[b, sl]
            d0 = hv + rv - tv
            d1 = hv2 + rv2 - tv2
            ad0 = ad0 + d0 * d0
            ad1 = ad1 + d1 * d1
            ah = ah + hv * hv
            at = at + tv * tv
            ahc = ahc + hv2 * hv2
            atc = atc + tv2 * tv2
            ar = ar + rv * rv
            arc = arc + rv2 * rv2

        pos_v = _sqrt_v(jnp.broadcast_to(jnp.sum(ad0), (_L,)))
        neg_v = _sqrt_v(jnp.broadcast_to(jnp.sum(ad1), (_L,)))
        loss_a = loss_a + jnp.maximum(pos_v - neg_v + 1.0, 0.0)

        ent_a = ent_a + _relu_bcast(jnp.sum(ah) - 1.0)
        ent_a = ent_a + _relu_bcast(jnp.sum(at) - 1.0)
        ent_a = ent_a + _relu_bcast(jnp.sum(ahc) - 1.0)
        ent_a = ent_a + _relu_bcast(jnp.sum(atc) - 1.0)
        rel_a = rel_a + _relu_bcast(jnp.sum(ar) - 1.0)
        rel_a = rel_a + _relu_bcast(jnp.sum(arc) - 1.0)
        return loss_a, ent_a, rel_a

    loss_a, ent_a, rel_a = lax.fori_loop(
        0, _NB, body, (zero, zero, zero))

    part[...] = (loss_a * (1.0 / _BATCH)
                 + ent_a * (1.0 / (4 * _BATCH))
                 + rel_a * (1.0 / (2 * _BATCH)))
    pltpu.sync_copy(part, out_hbm.at[wid])


@functools.partial(
    pl.kernel,
    out_type=jax.ShapeDtypeStruct((_NW, _L), jnp.float32),
    mesh=plsc.VectorSubcoreMesh(core_axis_name="c", subcore_axis_name="s"),
    compiler_params=pltpu.CompilerParams(needs_layout_passes=False),
    scratch_types=[
        pltpu.VMEM((_NB,), jnp.int32),
        pltpu.VMEM((_NB,), jnp.int32),
        pltpu.VMEM((_NB,), jnp.int32),
        pltpu.VMEM((_NB,), jnp.int32),
        pltpu.VMEM((_NB,), jnp.int32),
        pltpu.VMEM((_NB,), jnp.int32),
        pltpu.VMEM((_NB, _DIM), jnp.float32),
        pltpu.VMEM((_NB, _DIM), jnp.float32),
        pltpu.VMEM((_NB, _DIM), jnp.float32),
        pltpu.VMEM((_NB, _DIM), jnp.float32),
        pltpu.VMEM((_NB, _DIM), jnp.float32),
        pltpu.VMEM((_NB, _DIM), jnp.float32),
        pltpu.VMEM((_L,), jnp.float32),
        pltpu.SemaphoreType.DMA,
    ],
)
def _transe_sc(ent_hbm, rel_hbm, h_hbm, r_hbm, t_hbm, hc_hbm, rc_hbm, tc_hbm,
               out_hbm, *scratch):
    _tec_body(ent_hbm, rel_hbm, h_hbm, r_hbm, t_hbm, hc_hbm, rc_hbm, tc_hbm,
              out_hbm, *scratch)


@jax.jit
def kernel(ent_embedding, rel_embedding, current_triples, corrupted_triples):
    h = current_triples[:, 0]
    r = current_triples[:, 1]
    t = current_triples[:, 2]
    hc = corrupted_triples[:, 0]
    rc = corrupted_triples[:, 1]
    tc = corrupted_triples[:, 2]
    parts = _transe_sc(ent_embedding, rel_embedding, h, r, t, hc, rc, tc)
    return jnp.sum(parts) * (1.0 / _L)


# speedup vs baseline: 2.8705x; 1.0138x over previous
import functools

import jax
import jax.numpy as jnp
from jax import lax
from jax.experimental import pallas as pl
from jax.experimental.pallas import tpu as pltpu
from jax.experimental.pallas import tpu_sc as plsc

_BATCH = 4096
_DIM = 128
_L = 16

_info = plsc.get_sparse_core_info()
_NC = _info.num_cores
_NS = _info.num_subcores
_NW = _NC * _NS
_NB = _BATCH // _NW
_CH = _DIM // _L


def _sqrt_v(x):
    i = plsc.bitcast(x, jnp.int32)
    i = jnp.int32(0x5F3759DF) - lax.shift_right_logical(i, 1)
    z = plsc.bitcast(i, jnp.float32)
    for _ in range(3):
        z = z * (1.5 - 0.5 * x * z * z)
    return x * z


def _allsum(v, perms):
    del perms
    return jnp.broadcast_to(jnp.sum(v), (_L,))


def _tec_body(ent_hbm, rel_hbm, h_hbm, r_hbm, t_hbm, hc_hbm, rc_hbm, tc_hbm,
              out_hbm,
              ih, ir, it, ihc, irc, itc,
              gh, gr, gt, ghc, grc, gtc,
              part, sem):
    wid = lax.axis_index("s") * _NC + lax.axis_index("c")
    base = wid * _NB

    pltpu.sync_copy(h_hbm.at[pl.ds(base, _NB)], ih)
    pltpu.sync_copy(r_hbm.at[pl.ds(base, _NB)], ir)
    pltpu.sync_copy(t_hbm.at[pl.ds(base, _NB)], it)
    pltpu.sync_copy(hc_hbm.at[pl.ds(base, _NB)], ihc)
    pltpu.sync_copy(rc_hbm.at[pl.ds(base, _NB)], irc)
    pltpu.sync_copy(tc_hbm.at[pl.ds(base, _NB)], itc)

    c0 = pltpu.async_copy(ent_hbm.at[ih], gh, sem)
    c1 = pltpu.async_copy(rel_hbm.at[ir], gr, sem)
    c2 = pltpu.async_copy(ent_hbm.at[it], gt, sem)
    c3 = pltpu.async_copy(ent_hbm.at[ihc], ghc, sem)
    c4 = pltpu.async_copy(rel_hbm.at[irc], grc, sem)
    c5 = pltpu.async_copy(ent_hbm.at[itc], gtc, sem)
    c0.wait(); c1.wait(); c2.wait(); c3.wait(); c4.wait(); c5.wait()

    zero = jnp.zeros((_L,), jnp.float32)
    lane = lax.iota(jnp.int32, _L)
    perms = [lax.bitwise_xor(lane, jnp.int32(1 << k)) for k in range(4)]

    def body(b, carry):
        loss_a, ent_a, rel_a = carry
        ad0 = zero; ad1 = zero
        ah = zero; at = zero; ahc = zero; atc = zero
        ar = zero; arc = zero
        for c in range(_CH):
            sl = pl.ds(c * _L, _L)
            hv = gh[b, sl]; rv = gr[b, sl]; tv = gt[b, sl]
            hv2 = ghc[b, sl]; rv2 = grc[b, sl]; tv2 = gtc[b, sl]
            d0 = hv + rv - tv
            d1 = hv2 + rv2 - tv2
            ad0 = ad0 + d0 * d0
            ad1 = ad1 + d1 * d1
            ah = ah + hv * hv
            at = at + tv * tv
            ahc = ahc + hv2 * hv2
            atc = atc + tv2 * tv2
            ar = ar + rv * rv
            arc = arc + rv2 * rv2

        pos_v = _sqrt_v(_allsum(ad0, perms))
        neg_v = _sqrt_v(_allsum(ad1, perms))
        loss_a = loss_a + jnp.maximum(pos_v - neg_v + 1.0, 0.0)

        ent_a = ent_a + jnp.maximum(_allsum(ah, perms) - 1.0, 0.0)
        ent_a = ent_a + jnp.maximum(_allsum(at, perms) - 1.0, 0.0)
        ent_a = ent_a + jnp.maximum(_allsum(ahc, perms) - 1.0, 0.0)
        ent_a = ent_a + jnp.maximum(_allsum(atc, perms) - 1.0, 0.0)
        rel_a = rel_a + jnp.maximum(_allsum(ar, perms) - 1.0, 0.0)
        rel_a = rel_a + jnp.maximum(_allsum(arc, perms) - 1.0, 0.0)
        return loss_a, ent_a, rel_a

    loss_a, ent_a, rel_a = lax.fori_loop(
        0, _NB, body, (zero, zero, zero))

    part[...] = (loss_a * (1.0 / _BATCH)
                 + ent_a * (1.0 / (4 * _BATCH))
                 + rel_a * (1.0 / (2 * _BATCH)))
    pltpu.sync_copy(part, out_hbm.at[wid])


@functools.partial(
    pl.kernel,
    out_type=jax.ShapeDtypeStruct((_NW, _L), jnp.float32),
    mesh=plsc.VectorSubcoreMesh(core_axis_name="c", subcore_axis_name="s"),
    compiler_params=pltpu.CompilerParams(needs_layout_passes=False),
    scratch_types=[
        pltpu.VMEM((_NB,), jnp.int32),
        pltpu.VMEM((_NB,), jnp.int32),
        pltpu.VMEM((_NB,), jnp.int32),
        pltpu.VMEM((_NB,), jnp.int32),
        pltpu.VMEM((_NB,), jnp.int32),
        pltpu.VMEM((_NB,), jnp.int32),
        pltpu.VMEM((_NB, _DIM), jnp.float32),
        pltpu.VMEM((_NB, _DIM), jnp.float32),
        pltpu.VMEM((_NB, _DIM), jnp.float32),
        pltpu.VMEM((_NB, _DIM), jnp.float32),
        pltpu.VMEM((_NB, _DIM), jnp.float32),
        pltpu.VMEM((_NB, _DIM), jnp.float32),
        pltpu.VMEM((_L,), jnp.float32),
        pltpu.SemaphoreType.DMA,
    ],
)
def _transe_sc(ent_hbm, rel_hbm, h_hbm, r_hbm, t_hbm, hc_hbm, rc_hbm, tc_hbm,
               out_hbm, *scratch):
    _tec_body(ent_hbm, rel_hbm, h_hbm, r_hbm, t_hbm, hc_hbm, rc_hbm, tc_hbm,
              out_hbm, *scratch)


@jax.jit
def kernel(ent_embedding, rel_embedding, current_triples, corrupted_triples):
    h = current_triples[:, 0]
    r = current_triples[:, 1]
    t = current_triples[:, 2]
    hc = corrupted_triples[:, 0]
    rc = corrupted_triples[:, 1]
    tc = corrupted_triples[:, 2]
    parts = _transe_sc(ent_embedding, rel_embedding, h, r, t, hc, rc, tc)
    return jnp.sum(parts) * (1.0 / _L)
